# baseline (speedup 1.0000x reference)
import functools

import jax
import jax.numpy as jnp
from jax import lax
from jax.experimental import pallas as pl
from jax.experimental.pallas import tpu as pltpu
from jax.experimental.pallas import tpu_sc as plsc

D = 64
BATCH = 4096
HIST = 200
B = BATCH * HIST
NC, NS = 2, 16
NW = NC * NS
BPW = B // NW
CH = 128
NCHUNK = BPW // CH
NBUF = 4
LOOK = 2
NROUND = NCHUNK // NBUF

_mesh = plsc.VectorSubcoreMesh(core_axis_name="c", subcore_axis_name="s")


@functools.partial(
    pl.kernel,
    mesh=_mesh,
    compiler_params=pltpu.CompilerParams(use_tc_tiling_on_sc=False),
    out_type=jax.ShapeDtypeStruct((B, D), jnp.float32),
    scratch_types=(
        [
            pltpu.VMEM((NCHUNK, CH), jnp.int32),
            pltpu.VMEM((NBUF, CH, D), jnp.float32),
        ]
        + [pltpu.SemaphoreType.DMA] * (2 * NBUF)
    ),
)
def _embed(table_hbm, idx_hbm, out_hbm, idx_v, bufs, *sems):
  gsem = sems[:NBUF]
  ssem = sems[NBUF:]
  wid = lax.axis_index("s") * NC + lax.axis_index("c")
  base = wid * BPW

  pltpu.sync_copy(idx_hbm.at[wid], idx_v)

  def start_gather(c, b):
    pltpu.async_copy(table_hbm.at[idx_v.at[c]], bufs.at[b], gsem[b])

  def wait_gather(c, b):
    pltpu.make_async_copy(table_hbm.at[idx_v.at[c]], bufs.at[b], gsem[b]).wait()

  def start_scatter(c, b):
    pltpu.async_copy(bufs.at[b], out_hbm.at[pl.ds(base + c * CH, CH)], ssem[b])

  def wait_scatter(b):
    pltpu.make_async_copy(bufs.at[b], out_hbm.at[pl.ds(base, CH)], ssem[b]).wait()

  for b in range(LOOK):
    start_gather(b, b)

  def round_body(r, carry):
    for b in range(NBUF):
      c = r * NBUF + b
      wait_gather(c, b)
      start_scatter(c, b)
      c2 = c + LOOK
      b2 = (b + LOOK) % NBUF

      @pl.when(c2 < NCHUNK)
      def _():
        @pl.when(c2 >= NBUF)
        def _():
          wait_scatter(b2)

        start_gather(c2, b2)

    return carry

  lax.fori_loop(0, NROUND, round_body, 0)

  for b in range(NBUF):
    wait_scatter(b)


def kernel(input, table):
  idx = input.reshape(NW, NCHUNK, CH)
  out = _embed(table, idx)
  return out.reshape(BATCH, HIST, D)


# speedup vs baseline: 1.0052x; 1.0052x over previous
import functools

import jax
import jax.numpy as jnp
from jax import lax
from jax.experimental import pallas as pl
from jax.experimental.pallas import tpu as pltpu
from jax.experimental.pallas import tpu_sc as plsc

D = 64
BATCH = 4096
HIST = 200
B = BATCH * HIST
NC, NS = 2, 16
NW = NC * NS
BPW = B // NW
CH = 256
NCHUNK = BPW // CH
NBUF = 4
LOOK = 2
NROUND = NCHUNK // NBUF

_mesh = plsc.VectorSubcoreMesh(core_axis_name="c", subcore_axis_name="s")


@functools.partial(
    pl.kernel,
    mesh=_mesh,
    compiler_params=pltpu.CompilerParams(use_tc_tiling_on_sc=False),
    out_type=jax.ShapeDtypeStruct((B, D), jnp.float32),
    scratch_types=(
        [
            pltpu.VMEM((NCHUNK, CH), jnp.int32),
            pltpu.VMEM((NBUF, CH, D), jnp.float32),
        ]
        + [pltpu.SemaphoreType.DMA] * (2 * NBUF)
    ),
)
def _embed(table_hbm, idx_hbm, out_hbm, idx_v, bufs, *sems):
  gsem = sems[:NBUF]
  ssem = sems[NBUF:]
  wid = lax.axis_index("s") * NC + lax.axis_index("c")
  base = wid * BPW

  pltpu.sync_copy(idx_hbm.at[wid], idx_v)

  def start_gather(c, b):
    pltpu.async_copy(table_hbm.at[idx_v.at[c]], bufs.at[b], gsem[b])

  def wait_gather(c, b):
    pltpu.make_async_copy(table_hbm.at[idx_v.at[c]], bufs.at[b], gsem[b]).wait()

  def start_scatter(c, b):
    pltpu.async_copy(bufs.at[b], out_hbm.at[pl.ds(base + c * CH, CH)], ssem[b])

  def wait_scatter(b):
    pltpu.make_async_copy(bufs.at[b], out_hbm.at[pl.ds(base, CH)], ssem[b]).wait()

  for b in range(LOOK):
    start_gather(b, b)

  def round_body(r, carry):
    for b in range(NBUF):
      c = r * NBUF + b
      wait_gather(c, b)
      start_scatter(c, b)
      c2 = c + LOOK
      b2 = (b + LOOK) % NBUF

      @pl.when(c2 < NCHUNK)
      def _():
        @pl.when(c2 >= NBUF)
        def _():
          wait_scatter(b2)

        start_gather(c2, b2)

    return carry

  lax.fori_loop(0, NROUND, round_body, 0)

  for b in range(NBUF):
    wait_scatter(b)


def kernel(input, table):
  idx = input.reshape(NW, NCHUNK, CH)
  out = _embed(table, idx)
  return out.reshape(BATCH, HIST, D)
